# Initial kernel scaffold; baseline (speedup 1.0000x reference)
import jax
import jax.numpy as jnp
from jax.experimental import pallas as pl


def kernel(distance_to_center, edge_index, h_gru, gru_inp, beta, W1, b1, W2, b2):
    raise NotImplementedError("write your pallas kernel here")


# speedup vs baseline: 58.9433x; 58.9433x over previous
import functools

import jax
import jax.numpy as jnp
from jax import lax
from jax.experimental import pallas as pl
from jax.experimental.pallas import tpu as pltpu
from jax.experimental.pallas import tpu_sc as plsc

N = 50000
E = 1600000
NW = 32
EPT = E // NW
CH = 2000
NCH = EPT // CH
GPC = CH // 16
NB = 25
BR = N // NB


def _vtake(x, idx):
    dnums = lax.GatherDimensionNumbers(
        offset_dims=(), collapsed_slice_dims=(0,), start_index_map=(0,))
    return lax.gather(x, idx[:, None], dnums, (1,),
                      mode=lax.GatherScatterMode.PROMISE_IN_BOUNDS)


def _sc_body(edge_hbm, dist_hbm, out_hbm, dtab, acc, sbuf, dbuf, sem):
    cid = lax.axis_index("c")
    sid = lax.axis_index("s")
    wid = sid * 2 + cid
    base = wid * EPT

    pltpu.sync_copy(dist_hbm, dtab)

    inf16 = jnp.full((16,), 3e38, dtype=jnp.float32)

    def init_body(i, _):
        acc[pl.ds(i * 16, 16)] = inf16
        return 0

    lax.fori_loop(0, N // 16, init_body, 0)

    iota = lax.iota(jnp.int32, 16)

    def group_body(g, _):
        s = sbuf[pl.ds(g * 16, 16)]
        d = dbuf[pl.ds(g * 16, 16)]
        m = plsc.load_gather(dtab, [s])
        d_s, m_s = plsc.sort_key_val(d, m)
        for o in (1, 2, 4, 8):
            j = jnp.maximum(iota - o, 0)
            pd = _vtake(d_s, j)
            pm = _vtake(m_s, j)
            take = (iota >= o) & (pd == d_s)
            m_s = jnp.where(take, jnp.minimum(m_s, pm), m_s)
        nd = _vtake(d_s, jnp.minimum(iota + 1, 15))
        last = (iota == 15) | (nd != d_s)
        old = plsc.load_gather(acc, [d_s])
        plsc.store_scatter(acc, [d_s], jnp.minimum(old, m_s), mask=last)
        return 0

    def chunk_body(c, _):
        off = base + c * CH
        pltpu.sync_copy(edge_hbm.at[pl.ds(off, CH)], sbuf)
        pltpu.sync_copy(edge_hbm.at[pl.ds(E + off, CH)], dbuf)
        lax.fori_loop(0, GPC, group_body, 0)
        return 0

    lax.fori_loop(0, NCH, chunk_body, 0)

    def out_body(c, _):
        pltpu.sync_copy(acc.at[pl.ds(c * BR, BR)],
                        out_hbm.at[pl.ds((c * NW + wid) * BR, BR)])
        return 0

    lax.fori_loop(0, NB, out_body, 0)


def _sc_segment_min(edge_index, distance_to_center):
    mesh = plsc.VectorSubcoreMesh(core_axis_name="c", subcore_axis_name="s")
    kern = pl.kernel(
        _sc_body,
        out_type=jax.ShapeDtypeStruct((NB * NW * BR,), jnp.float32),
        mesh=mesh,
        scratch_types=[
            pltpu.VMEM((N,), jnp.float32),
            pltpu.VMEM((N,), jnp.float32),
            pltpu.VMEM((CH,), jnp.int32),
            pltpu.VMEM((CH,), jnp.int32),
            pltpu.SemaphoreType.DMA,
        ],
        compiler_params=pltpu.CompilerParams(needs_layout_passes=False),
    )
    return kern(edge_index.reshape(2 * E),
                distance_to_center).reshape(NB, NW, BR)


def _mlp_body(hg_ref, gi_ref, w1_ref, b1_ref, w2_ref, b2_ref, temp_ref):
    x = jnp.concatenate([hg_ref[...], gi_ref[...]], axis=1)
    h1 = jnp.dot(x, w1_ref[...], preferred_element_type=jnp.float32)
    h1 = h1 + b1_ref[...]
    h1 = jnp.where(h1 > 0, h1, jnp.exp(h1) - 1.0)
    tl = jnp.dot(h1, w2_ref[...], preferred_element_type=jnp.float32)
    t = jax.nn.sigmoid(tl + b2_ref[...])
    temp_ref[0, 0, :] = t[:, 0]


def _mlp(h_gru, gru_inp, W1, b1, W2, b2):
    b1r = b1.reshape(1, 256)
    b2r = b2.reshape(1, 1)
    return pl.pallas_call(
        _mlp_body,
        grid=(NB,),
        in_specs=[
            pl.BlockSpec((BR, 256), lambda i: (i, 0)),
            pl.BlockSpec((BR, 256), lambda i: (i, 0)),
            pl.BlockSpec((512, 256), lambda i: (0, 0)),
            pl.BlockSpec((1, 256), lambda i: (0, 0)),
            pl.BlockSpec((256, 1), lambda i: (0, 0)),
            pl.BlockSpec((1, 1), lambda i: (0, 0)),
        ],
        out_specs=pl.BlockSpec((1, 1, BR), lambda i: (i, 0, 0)),
        out_shape=jax.ShapeDtypeStruct((NB, 1, BR), jnp.float32),
    )(h_gru, gru_inp, W1, b1r, W2, b2r)


def _combine_body(part_ref, temp_ref, beta_ref, ds_ref, ts_ref, sw_ref,
                  lab_ref):
    md = jnp.min(part_ref[0], axis=0)
    md = jnp.minimum(md, 1e4)
    bw = beta_ref[0, 0] * beta_ref[0, 0]
    ds = jnp.exp(-bw * md)
    t = temp_ref[0, 0, :]
    sw = ds * t
    ds_ref[0, 0, :] = ds
    ts_ref[0, 0, :] = t
    sw_ref[0, 0, :] = sw
    lab_ref[0, 0, :] = jnp.where(sw >= 0.5, 1.0, 0.0)


def _combine(partial, temp, beta):
    blk = pl.BlockSpec((1, 1, BR), lambda i: (i, 0, 0))
    shp = jax.ShapeDtypeStruct((NB, 1, BR), jnp.float32)
    return pl.pallas_call(
        _combine_body,
        grid=(NB,),
        in_specs=[
            pl.BlockSpec((1, NW, BR), lambda i: (i, 0, 0)),
            blk,
            pl.BlockSpec((1, 1), lambda i: (0, 0)),
        ],
        out_specs=[blk, blk, blk, blk],
        out_shape=[shp, shp, shp, shp],
    )(partial, temp, beta)


def kernel(distance_to_center, edge_index, h_gru, gru_inp, beta, W1, b1, W2,
           b2):
    partial = _sc_segment_min(edge_index, distance_to_center)
    temp = _mlp(h_gru, gru_inp, W1, b1, W2, b2)
    ds, ts, sw, lab = _combine(partial, temp, beta)
    r = lambda x: x.reshape(N)
    return r(ds), r(ts), r(sw), r(lab)


# speedup vs baseline: 68.6099x; 1.1640x over previous
import functools

import jax
import jax.numpy as jnp
from jax import lax
from jax.experimental import pallas as pl
from jax.experimental.pallas import tpu as pltpu
from jax.experimental.pallas import tpu_sc as plsc

N = 50000
E = 1600000
NW = 32
EPT = E // NW
CH = 2000
NCH = EPT // CH
GPC = CH // 16
NB = 25
BR = N // NB


def _vtake(x, idx):
    dnums = lax.GatherDimensionNumbers(
        offset_dims=(), collapsed_slice_dims=(0,), start_index_map=(0,))
    return lax.gather(x, idx[:, None], dnums, (1,),
                      mode=lax.GatherScatterMode.PROMISE_IN_BOUNDS)


def _sc_body(edge_hbm, dist_hbm, out_hbm, dtab, acc, sbuf0, dbuf0, sbuf1,
             dbuf1, sem0, sem1):
    cid = lax.axis_index("c")
    sid = lax.axis_index("s")
    wid = sid * 2 + cid
    base = wid * EPT

    sbufs = (sbuf0, sbuf1)
    dbufs = (dbuf0, dbuf1)
    sems = (sem0, sem1)

    def fire(c, b):
        off = base + c * CH
        sd = pltpu.async_copy(edge_hbm.at[pl.ds(off, CH)], sbufs[b], sems[b])
        dd = pltpu.async_copy(edge_hbm.at[pl.ds(E + off, CH)], dbufs[b],
                              sems[b])
        return sd, dd

    pend = [fire(0, 0), None]

    pltpu.sync_copy(dist_hbm, dtab)

    inf16 = jnp.full((16,), 3e38, dtype=jnp.float32)

    def init_body(i, _):
        acc[pl.ds(i * 16, 16)] = inf16
        return 0

    lax.fori_loop(0, N // 16, init_body, 0)

    iota = lax.iota(jnp.int32, 16)

    def make_group_body(sbuf, dbuf):
        def group_body(g, _):
            s = sbuf[pl.ds(g * 16, 16)]
            d = dbuf[pl.ds(g * 16, 16)]
            m = plsc.load_gather(dtab, [s])
            d_s, m_s = plsc.sort_key_val(d, m)
            for o in (1, 2, 4, 8):
                j = jnp.maximum(iota - o, 0)
                pd = _vtake(d_s, j)
                pm = _vtake(m_s, j)
                take = (iota >= o) & (pd == d_s)
                m_s = jnp.where(take, jnp.minimum(m_s, pm), m_s)
            nd = _vtake(d_s, jnp.minimum(iota + 1, 15))
            last = (iota == 15) | (nd != d_s)
            old = plsc.load_gather(acc, [d_s])
            plsc.store_scatter(acc, [d_s], jnp.minimum(old, m_s), mask=last)
            return 0

        return group_body

    for c in range(NCH):
        b = c & 1
        if c + 1 < NCH:
            pend[(c + 1) & 1] = fire(c + 1, (c + 1) & 1)
        pend[b][0].wait()
        pend[b][1].wait()
        lax.fori_loop(0, GPC, make_group_body(sbufs[b], dbufs[b]), 0)

    def out_body(c, _):
        pltpu.sync_copy(acc.at[pl.ds(c * BR, BR)],
                        out_hbm.at[pl.ds((c * NW + wid) * BR, BR)])
        return 0

    lax.fori_loop(0, NB, out_body, 0)


def _sc_segment_min(edge_index, distance_to_center):
    mesh = plsc.VectorSubcoreMesh(core_axis_name="c", subcore_axis_name="s")
    kern = pl.kernel(
        _sc_body,
        out_type=jax.ShapeDtypeStruct((NB * NW * BR,), jnp.float32),
        mesh=mesh,
        scratch_types=[
            pltpu.VMEM((N,), jnp.float32),
            pltpu.VMEM((N,), jnp.float32),
            pltpu.VMEM((CH,), jnp.int32),
            pltpu.VMEM((CH,), jnp.int32),
            pltpu.VMEM((CH,), jnp.int32),
            pltpu.VMEM((CH,), jnp.int32),
            pltpu.SemaphoreType.DMA,
            pltpu.SemaphoreType.DMA,
        ],
        compiler_params=pltpu.CompilerParams(needs_layout_passes=False),
    )
    return kern(edge_index.reshape(2 * E),
                distance_to_center).reshape(NB, NW, BR)


def _mlp_body(hg_ref, gi_ref, w1_ref, b1_ref, w2_ref, b2_ref, temp_ref):
    x = jnp.concatenate([hg_ref[...], gi_ref[...]], axis=1)
    h1 = jnp.dot(x, w1_ref[...], preferred_element_type=jnp.float32)
    h1 = h1 + b1_ref[...]
    h1 = jnp.where(h1 > 0, h1, jnp.exp(h1) - 1.0)
    tl = jnp.dot(h1, w2_ref[...], preferred_element_type=jnp.float32)
    t = jax.nn.sigmoid(tl + b2_ref[...])
    temp_ref[0, 0, :] = t[:, 0]


def _mlp(h_gru, gru_inp, W1, b1, W2, b2):
    b1r = b1.reshape(1, 256)
    b2r = b2.reshape(1, 1)
    return pl.pallas_call(
        _mlp_body,
        grid=(NB,),
        in_specs=[
            pl.BlockSpec((BR, 256), lambda i: (i, 0)),
            pl.BlockSpec((BR, 256), lambda i: (i, 0)),
            pl.BlockSpec((512, 256), lambda i: (0, 0)),
            pl.BlockSpec((1, 256), lambda i: (0, 0)),
            pl.BlockSpec((256, 1), lambda i: (0, 0)),
            pl.BlockSpec((1, 1), lambda i: (0, 0)),
        ],
        out_specs=pl.BlockSpec((1, 1, BR), lambda i: (i, 0, 0)),
        out_shape=jax.ShapeDtypeStruct((NB, 1, BR), jnp.float32),
    )(h_gru, gru_inp, W1, b1r, W2, b2r)


def _combine_body(part_ref, temp_ref, beta_ref, ds_ref, ts_ref, sw_ref,
                  lab_ref):
    md = jnp.min(part_ref[0], axis=0)
    md = jnp.minimum(md, 1e4)
    bw = beta_ref[0, 0] * beta_ref[0, 0]
    ds = jnp.exp(-bw * md)
    t = temp_ref[0, 0, :]
    sw = ds * t
    ds_ref[0, 0, :] = ds
    ts_ref[0, 0, :] = t
    sw_ref[0, 0, :] = sw
    lab_ref[0, 0, :] = jnp.where(sw >= 0.5, 1.0, 0.0)


def _combine(partial, temp, beta):
    blk = pl.BlockSpec((1, 1, BR), lambda i: (i, 0, 0))
    shp = jax.ShapeDtypeStruct((NB, 1, BR), jnp.float32)
    return pl.pallas_call(
        _combine_body,
        grid=(NB,),
        in_specs=[
            pl.BlockSpec((1, NW, BR), lambda i: (i, 0, 0)),
            blk,
            pl.BlockSpec((1, 1), lambda i: (0, 0)),
        ],
        out_specs=[blk, blk, blk, blk],
        out_shape=[shp, shp, shp, shp],
    )(partial, temp, beta)


def kernel(distance_to_center, edge_index, h_gru, gru_inp, beta, W1, b1, W2,
           b2):
    temp = _mlp(h_gru, gru_inp, W1, b1, W2, b2)
    partial = _sc_segment_min(edge_index, distance_to_center)
    ds, ts, sw, lab = _combine(partial, temp, beta)
    r = lambda x: x.reshape(N)
    return r(ds), r(ts), r(sw), r(lab)
